# baseline (speedup 1.0000x reference)
import functools

import jax
import jax.numpy as jnp
from jax import lax
from jax.experimental import pallas as pl
from jax.experimental.pallas import tpu as pltpu
from jax.experimental.pallas import tpu_sc as plsc

_NC = 2
_NS = 16
_NW = _NC * _NS

_GCHUNK = 256


@functools.lru_cache(maxsize=None)
def _make_gather(B, D):
    b_per_w = B // _NW
    n_chunks = b_per_w // _GCHUNK
    assert n_chunks * _GCHUNK == b_per_w and n_chunks >= 4
    mesh = plsc.VectorSubcoreMesh(core_axis_name="c", subcore_axis_name="s")

    @functools.partial(
        pl.kernel,
        out_type=jax.ShapeDtypeStruct((B, 128), jnp.float32),
        mesh=mesh,
        scratch_types=[
            pltpu.VMEM((_GCHUNK,), jnp.int32),
            pltpu.VMEM((_GCHUNK,), jnp.int32),
            pltpu.VMEM((_GCHUNK,), jnp.int32),
            pltpu.VMEM((_GCHUNK, 128), jnp.float32),
            pltpu.VMEM((_GCHUNK, 128), jnp.float32),
            pltpu.VMEM((_GCHUNK, 128), jnp.float32),
            pltpu.SemaphoreType.DMA,
            pltpu.SemaphoreType.DMA,
            pltpu.SemaphoreType.DMA,
            pltpu.SemaphoreType.DMA,
            pltpu.SemaphoreType.DMA,
            pltpu.SemaphoreType.DMA,
            pltpu.SemaphoreType.DMA,
        ],
        compiler_params=pltpu.CompilerParams(
            use_tc_tiling_on_sc=True, needs_layout_passes=False),
    )
    def gather_kernel(idx_hbm, tab_hbm, out_hbm, idx_v0, idx_v1, idx_v2,
                      rows_v0, rows_v1, rows_v2,
                      si0, si1, si2, sg0, sg1, sg2, sem_s):
        idx_v = (idx_v0, idx_v1, idx_v2)
        sem_i = (si0, si1, si2)
        rows_v = (rows_v0, rows_v1, rows_v2)
        sem_g = (sg0, sg1, sg2)
        wid = lax.axis_index("s") * _NC + lax.axis_index("c")
        base_w = wid * b_per_w

        def load_idx(g, b):
            pltpu.async_copy(
                idx_hbm.at[pl.ds(base_w + g * _GCHUNK, _GCHUNK)],
                idx_v[b], sem_i[b])

        def wait_idx(b):
            pltpu.make_async_copy(
                idx_hbm.at[pl.ds(0, _GCHUNK)], idx_v[b], sem_i[b]).wait()

        def gather(b):
            pltpu.async_copy(tab_hbm.at[idx_v[b]], rows_v[b], sem_g[b])

        def wait_gather(b):
            pltpu.make_async_copy(
                tab_hbm.at[idx_v[b]], rows_v[b], sem_g[b]).wait()

        def store(g, b):
            pltpu.async_copy(
                rows_v[b],
                out_hbm.at[pl.ds(base_w + g * _GCHUNK, _GCHUNK), :],
                sem_s)

        def wait_store(b):
            pltpu.make_async_copy(
                rows_v[b], out_hbm.at[pl.ds(0, _GCHUNK), :], sem_s).wait()

        load_idx(0, 0)
        load_idx(1, 1)
        load_idx(2, 2)
        wait_idx(0)
        gather(0)

        def body(g, carry):
            b = lax.rem(g, 3)

            for r in range(3):
                @pl.when(b == r)
                def _():
                    wait_gather(r)
                    store(g, r)
                    wait_store(r)

                    @pl.when(g + 3 < n_chunks)
                    def _():
                        load_idx(g + 3, r)
                        wait_idx(r)
                        gather(r)
            return carry

        wait_idx(1)
        gather(1)
        wait_idx(2)
        gather(2)
        lax.fori_loop(0, n_chunks, body, 0)

    return gather_kernel


def kernel(x, weight):
    shape = x.shape
    B = x.size
    V, D = weight.shape
    flat_idx = jnp.reshape(x.astype(jnp.int32), (B,))
    tab = jnp.pad(weight, ((0, 0), (0, 128 - D)))
    out = _make_gather(B, D)(flat_idx, tab)
    return jnp.reshape(out[:, :D], shape + (D,))
